# baseline (speedup 1.0000x reference)
import functools

import jax
import jax.numpy as jnp
from jax import lax
from jax.experimental import pallas as pl
from jax.experimental.pallas import tpu as pltpu
from jax.experimental.pallas import tpu_sc as plsc

_N_FIELDS = 26
_VOCAB = 100000
_EDIM = 32
_BATCH = 16384
_N = _BATCH * _N_FIELDS
_NW = 32
_PER_W = _N // _NW
_CHUNK = 1664
_NCHUNK = _PER_W // _CHUNK
_PERIOD = 208
_LANES = 16

_mesh = plsc.VectorSubcoreMesh(core_axis_name="c", subcore_axis_name="s")


@functools.partial(
    pl.kernel,
    mesh=_mesh,
    out_type=jax.ShapeDtypeStruct((_N, _EDIM), jnp.float32),
    compiler_params=pltpu.CompilerParams(use_tc_tiling_on_sc=False),
    scratch_types=[
        pltpu.VMEM((_PERIOD,), jnp.int32),
        pltpu.VMEM((_CHUNK,), jnp.int32),
        pltpu.VMEM((_CHUNK,), jnp.int32),
        pltpu.VMEM((_CHUNK, _EDIM), jnp.float32),
        pltpu.SemaphoreType.DMA,
    ],
)
def _mk_gather(x_hbm, tab_hbm, out_hbm, off_v, xv, idxv, rows_v, sem):
    wid = lax.axis_index("s") * 2 + lax.axis_index("c")
    wbase = wid * _PER_W

    for j in range(_PERIOD // _LANES):
        lanes = lax.iota(jnp.int32, _LANES) + (j * _LANES)
        off_v[pl.ds(j * _LANES, _LANES)] = lax.rem(lanes, _N_FIELDS) * _VOCAB

    def chunk_body(c, carry):
        base = wbase + c * _CHUNK
        pltpu.sync_copy(x_hbm.at[pl.ds(base, _CHUNK)], xv)

        def add_off(j, carry2):
            p = lax.rem(j, _PERIOD // _LANES) * _LANES
            idxv[pl.ds(j * _LANES, _LANES)] = (
                xv[pl.ds(j * _LANES, _LANES)] + off_v[pl.ds(p, _LANES)]
            )
            return carry2

        lax.fori_loop(0, _CHUNK // _LANES, add_off, 0)

        pltpu.async_copy(tab_hbm.at[idxv], rows_v, sem).wait()
        pltpu.sync_copy(rows_v, out_hbm.at[pl.ds(base, _CHUNK)])
        return carry

    lax.fori_loop(0, _NCHUNK, chunk_body, 0)


def kernel(x, tables):
    x_flat = x.reshape(_N)
    tab_flat = tables.reshape(_N_FIELDS * _VOCAB, _EDIM)
    out = _mk_gather(x_flat, tab_flat)
    return out.reshape(_BATCH, _N_FIELDS * _EDIM)


# speedup vs baseline: 3.6404x; 3.6404x over previous
import functools

import jax
import jax.numpy as jnp
from jax import lax
from jax.experimental import pallas as pl
from jax.experimental.pallas import tpu as pltpu
from jax.experimental.pallas import tpu_sc as plsc

_N_FIELDS = 26
_VOCAB = 100000
_EDIM = 32
_BATCH = 16384
_NW = 32
_NPLANES = _N_FIELDS * _EDIM
_PLANES_PER_W = _NPLANES // _NW
_LANES = 16
_UNROLL = 8
_BCHUNK = 4096

_mesh = plsc.VectorSubcoreMesh(core_axis_name="c", subcore_axis_name="s")


@functools.partial(
    pl.kernel,
    mesh=_mesh,
    out_type=jax.ShapeDtypeStruct((_NPLANES, _BATCH), jnp.float32),
    compiler_params=pltpu.CompilerParams(
        use_tc_tiling_on_sc=True, needs_layout_passes=False
    ),
    scratch_types=[
        pltpu.VMEM((_VOCAB,), jnp.float32),
        pltpu.VMEM((_BCHUNK,), jnp.int32),
        pltpu.VMEM((_BCHUNK,), jnp.float32),
    ],
)
def _mk_gather(xt_hbm, tt_hbm, out_hbm, plane_v, xv, row_v):
    wid = lax.axis_index("s") * 2 + lax.axis_index("c")

    def do_plane(j, carry):
        c = wid * _PLANES_PER_W + j
        f = c // _EDIM
        e = lax.rem(c, _EDIM)
        pltpu.sync_copy(tt_hbm.at[f, e], plane_v)

        def do_bchunk(b, carry2):
            b0 = b * _BCHUNK
            pltpu.sync_copy(xt_hbm.at[f, pl.ds(b0, _BCHUNK)], xv)

            def gather_group(i, carry3):
                base = i * (_LANES * _UNROLL)
                for k in range(_UNROLL):
                    o = base + k * _LANES
                    idx = xv[pl.ds(o, _LANES)]
                    row_v[pl.ds(o, _LANES)] = plsc.load_gather(plane_v, [idx])
                return carry3

            lax.fori_loop(0, _BCHUNK // (_LANES * _UNROLL), gather_group, 0)
            pltpu.sync_copy(row_v, out_hbm.at[c, pl.ds(b0, _BCHUNK)])
            return carry2

        lax.fori_loop(0, _BATCH // _BCHUNK, do_bchunk, 0)
        return carry

    lax.fori_loop(0, _PLANES_PER_W, do_plane, 0)


def kernel(x, tables):
    xt = x.T
    tt = tables.transpose(0, 2, 1)
    out = _mk_gather(xt, tt)
    return out.T
